# Initial kernel scaffold; baseline (speedup 1.0000x reference)
import jax
import jax.numpy as jnp
from jax.experimental import pallas as pl


def kernel(J_msg, b, msg_node, idx_msg_edge, params):
    raise NotImplementedError("write your pallas kernel here")


# speedup vs baseline: 1.8859x; 1.8859x over previous
import jax
import jax.numpy as jnp
from jax.experimental import pallas as pl
from jax.experimental.pallas import tpu as pltpu

H = 64
NUM_PROP = 3


def _row_spec(rows, cols):
    return pl.BlockSpec((rows, cols), lambda i: (i, 0))


def _full_spec(shape):
    return pl.BlockSpec(shape, lambda i: (0,) * len(shape))


def _bdot(a, b):
    return jnp.dot(a.astype(jnp.bfloat16), b,
                   preferred_element_type=jnp.float32)


def _prep_body(bi_ref, bo_ref, jm_ref, av_ref, dv_ref, ffa_ref, ffb_ref):
    bi = bi_ref[...].astype(jnp.bfloat16).astype(jnp.float32)
    bo = bo_ref[...].astype(jnp.bfloat16).astype(jnp.float32)
    jm = jm_ref[...].astype(jnp.bfloat16).astype(jnp.float32)
    av = av_ref[...]
    dv = dv_ref[...]
    ffa_ref[...] = bi * av[0:1, :] + bo * av[1:2, :] + jm * av[2:3, :]
    ffb_ref[...] = bi * dv[0:1, :] + bo * dv[1:2, :] + jm * dv[2:3, :] + dv[3:4, :]


def _prep_call(bi, bo, jm, av, dv, block=8000):
    E = bi.shape[0]
    block = min(block, E)
    grid = (E // block,)
    return pl.pallas_call(
        _prep_body,
        grid=grid,
        in_specs=[
            _row_spec(block, 1),
            _row_spec(block, 1),
            _row_spec(block, 1),
            _full_spec((3, H)),
            _full_spec((4, H)),
        ],
        out_specs=[_row_spec(block, H), _row_spec(block, H)],
        out_shape=[
            jax.ShapeDtypeStruct((E, H), jnp.float32),
            jax.ShapeDtypeStruct((E, H), jnp.float32),
        ],
    )(bi, bo, jm, av, dv)


def _mid_body(g1_ref, g2_ref, w2_ref, b2_ref, w3_ref, b3_ref, m_ref):
    h1 = jnp.maximum(g1_ref[...] + g2_ref[...], 0.0)
    h2 = jnp.maximum(_bdot(h1, w2_ref[...]) + b2_ref[...], 0.0)
    m_ref[...] = _bdot(h2, w3_ref[...]) + b3_ref[...]


def _mid_call(g1, g2, w2t, b2, w3t, b3, block=3200):
    E2 = g1.shape[0]
    grid = (E2 // block,)
    return pl.pallas_call(
        _mid_body,
        grid=grid,
        in_specs=[
            _row_spec(block, H),
            _row_spec(block, H),
            _full_spec((H, H)),
            _full_spec((1, H)),
            _full_spec((H, H)),
            _full_spec((1, H)),
        ],
        out_specs=_row_spec(block, H),
        out_shape=jax.ShapeDtypeStruct((E2, H), jnp.float32),
    )(g1, g2, w2t, b2, w3t, b3)


def _update_body(agg_ref, st_ref, ffa_ref, ffb_ref,
                 wih_ref, bih_ref, whh_ref, bhh_ref, wsi_ref, wso_ref,
                 st_out_ref, u_ref=None, v_ref=None):
    state = st_ref[...]
    gi = _bdot(agg_ref[...], wih_ref[...]) + bih_ref[...]
    gh = _bdot(state, whh_ref[...]) + bhh_ref[...]
    r = jax.nn.sigmoid(gi[:, 0:H] + gh[:, 0:H])
    z = jax.nn.sigmoid(gi[:, H:2 * H] + gh[:, H:2 * H])
    n = jnp.tanh(gi[:, 2 * H:3 * H] + r * gh[:, 2 * H:3 * H])
    new_state = (1.0 - z) * n + z * state
    st_out_ref[...] = new_state
    if u_ref is not None:
        u_ref[...] = _bdot(new_state, wsi_ref[...]) + ffa_ref[...]
        v_ref[...] = _bdot(new_state, wso_ref[...]) + ffb_ref[...]


def _update_call(agg2, state, ffa, ffb, wiht, bih, whht, bhh,
                 wsit, wsot, with_uv, block=3200):
    E = agg2.shape[0]
    grid = (E // block,)
    out_specs = [_row_spec(block, H)]
    out_shape = [jax.ShapeDtypeStruct((E, H), jnp.float32)]
    if with_uv:
        out_specs += [_row_spec(block, H), _row_spec(block, H)]
        out_shape += [jax.ShapeDtypeStruct((E, H), jnp.float32),
                      jax.ShapeDtypeStruct((E, H), jnp.float32)]
    return pl.pallas_call(
        _update_body,
        grid=grid,
        in_specs=[
            _row_spec(block, H),
            _row_spec(block, H),
            _row_spec(block, H),
            _row_spec(block, H),
            _full_spec((H, 3 * H)),
            _full_spec((1, 3 * H)),
            _full_spec((H, 3 * H)),
            _full_spec((1, 3 * H)),
            _full_spec((H, H)),
            _full_spec((H, H)),
        ],
        out_specs=out_specs,
        out_shape=out_shape,
    )(agg2, state, ffa, ffb, wiht, bih, whht, bhh, wsit, wsot)


def _out_body(sagg_ref, b_ref, wh_ref, wb_ref, b1_ref, w2_ref, b2_ref,
              w3_ref, b3_ref, o_ref):
    bcol = b_ref[...].astype(jnp.bfloat16).astype(jnp.float32)
    x1 = _bdot(sagg_ref[...], wh_ref[...]) + bcol * wb_ref[...] + b1_ref[...]
    x1 = jnp.maximum(x1, 0.0)
    x2 = jnp.maximum(_bdot(x1, w2_ref[...]) + b2_ref[...], 0.0)
    y = _bdot(x2, w3_ref[...]) + b3_ref[...]
    m = jnp.max(y, axis=1, keepdims=True)
    lse = jnp.log(jnp.sum(jnp.exp(y - m), axis=1, keepdims=True)) + m
    o_ref[...] = y - lse


def _out_call(sagg, b, wh, wb, b1, w2t, b2, w3t, b3, block=2000):
    N = sagg.shape[0]
    grid = (N // block,)
    return pl.pallas_call(
        _out_body,
        grid=grid,
        in_specs=[
            _row_spec(block, H),
            _row_spec(block, 1),
            _full_spec((H, H)),
            _full_spec((1, H)),
            _full_spec((1, H)),
            _full_spec((H, H)),
            _full_spec((1, H)),
            _full_spec((H, 2)),
            _full_spec((1, 2)),
        ],
        out_specs=_row_spec(block, 2),
        out_shape=jax.ShapeDtypeStruct((N, 2), jnp.float32),
    )(sagg, b, wh, wb, b1, w2t, b2, w3t, b3)


def _b16(x):
    return x.astype(jnp.bfloat16)


def kernel(J_msg, b, msg_node, idx_msg_edge, params):
    N = b.shape[0]
    E = msg_node.shape[0]
    p = params

    W1b = _b16(p['msg_W1'])
    W1f = W1b.astype(jnp.float32)
    wsit = W1b[:, 0:H].T
    wsot = W1b[:, H + 8:2 * H + 8].T
    Wfi = W1f[:, H:H + 8]
    Wfo = W1f[:, 2 * H + 8:2 * H + 16]
    av = jnp.stack([Wfi[:, 0] - Wfi[:, 1],
                    Wfi[:, 2] - Wfi[:, 3],
                    Wfi[:, 4] - Wfi[:, 5] - Wfi[:, 6] + Wfi[:, 7]])
    dv = jnp.stack([Wfo[:, 0] - Wfo[:, 1],
                    Wfo[:, 2] - Wfo[:, 3],
                    Wfo[:, 4] - Wfo[:, 5] - Wfo[:, 6] + Wfo[:, 7],
                    p['msg_b1']])
    w2t = _b16(p['msg_W2']).T
    b2 = p['msg_b2'].reshape(1, H)
    w3t = _b16(p['msg_W3']).T
    b3 = p['msg_b3'].reshape(1, H)
    wiht = _b16(p['gru_Wih']).T
    whht = _b16(p['gru_Whh']).T
    bih = p['gru_bih'].reshape(1, 3 * H)
    bhh = p['gru_bhh'].reshape(1, 3 * H)
    OW1 = _b16(p['out_W1'])
    wh = OW1[:, 0:H].T
    wb = (OW1[:, H].astype(jnp.float32)
          - OW1[:, H + 1].astype(jnp.float32)).reshape(1, H)
    ob1 = p['out_b1'].reshape(1, H)
    ow2t = _b16(p['out_W2']).T
    ob2 = p['out_b2'].reshape(1, H)
    ow3t = _b16(p['out_W3']).T
    ob3 = p['out_b3'].reshape(1, 2)

    mn0 = msg_node[:, 0]
    mn1 = msg_node[:, 1]
    ein = idx_msg_edge[:, 0]
    eout = idx_msg_edge[:, 1]

    bflat = b[:, 0]
    bi = bflat[mn0].reshape(E, 1)
    bo = bflat[mn1].reshape(E, 1)

    ffa, ffb = _prep_call(bi, bo, J_msg, av, dv)

    state = jnp.zeros((E, H), jnp.float32)
    U, V = ffa, ffb
    for step in range(NUM_PROP):
        g1 = U[ein]
        g2 = V[eout]
        m = _mid_call(g1, g2, w2t, b2, w3t, b3)
        agg2 = jnp.zeros((E, H), jnp.float32).at[eout].add(m)
        outs = _update_call(agg2, state, ffa, ffb, wiht, bih,
                            whht, bhh, wsit, wsot,
                            with_uv=(step < NUM_PROP - 1))
        if step < NUM_PROP - 1:
            state, U, V = outs
        else:
            state = outs[0]

    sagg = jnp.zeros((N, H), jnp.float32).at[mn1].add(state)
    return _out_call(sagg, b, wh, wb, ob1, ow2t, ob2, ow3t, ob3)


# speedup vs baseline: 11.4684x; 6.0810x over previous
import functools

import jax
import jax.numpy as jnp
from jax import lax
from jax.experimental import pallas as pl
from jax.experimental.pallas import tpu as pltpu
from jax.experimental.pallas import tpu_sc as plsc

H = 64
NUM_PROP = 3

_NC, _NS = 2, 16
_NW = _NC * _NS

_SC_MESH = dict(core_axis_name="c", subcore_axis_name="s")


def _wid():
    return lax.axis_index("s") * _NC + lax.axis_index("c")


def _row_spec(rows, cols):
    return pl.BlockSpec((rows, cols), lambda i: (i, 0))


def _full_spec(shape):
    return pl.BlockSpec(shape, lambda i: (0,) * len(shape))


def _bdot(a, b):
    return jnp.dot(a.astype(jnp.bfloat16), b,
                   preferred_element_type=jnp.float32)


def _prep_body(bi_ref, bo_ref, jm_ref, av_ref, dv_ref, fa_ref):
    bi = bi_ref[...].astype(jnp.bfloat16).astype(jnp.float32)
    bo = bo_ref[...].astype(jnp.bfloat16).astype(jnp.float32)
    jm = jm_ref[...].astype(jnp.bfloat16).astype(jnp.float32)
    av = av_ref[...]
    dv = dv_ref[...]
    ffa = bi * av[0:1, :] + bo * av[1:2, :] + jm * av[2:3, :]
    ffb = bi * dv[0:1, :] + bo * dv[1:2, :] + jm * dv[2:3, :] + dv[3:4, :]
    fa_ref[...] = jnp.concatenate([ffa, ffb], axis=1)


def _prep_call(bi, bo, jm, av, dv, block=8000):
    E = bi.shape[0]
    block = min(block, E)
    grid = (E // block,)
    return pl.pallas_call(
        _prep_body,
        grid=grid,
        in_specs=[
            _row_spec(block, 1),
            _row_spec(block, 1),
            _row_spec(block, 1),
            _full_spec((3, H)),
            _full_spec((4, H)),
        ],
        out_specs=_row_spec(block, 2 * H),
        out_shape=jax.ShapeDtypeStruct((E, 2 * H), jnp.float32),
    )(bi, bo, jm, av, dv)


def _mid_body(g1_ref, g2_ref, w2_ref, b2_ref, w3_ref, b3_ref, m_ref):
    h1 = jnp.maximum(g1_ref[:, 0:H] + g2_ref[:, H:2 * H], 0.0)
    h2 = jnp.maximum(_bdot(h1, w2_ref[...]) + b2_ref[...], 0.0)
    m_ref[...] = _bdot(h2, w3_ref[...]) + b3_ref[...]


def _mid_call(g1, g2, w2t, b2, w3t, b3, block=3200):
    E2 = g1.shape[0]
    grid = (E2 // block,)
    return pl.pallas_call(
        _mid_body,
        grid=grid,
        in_specs=[
            _row_spec(block, 2 * H),
            _row_spec(block, 2 * H),
            _full_spec((H, H)),
            _full_spec((1, H)),
            _full_spec((H, H)),
            _full_spec((1, H)),
        ],
        out_specs=_row_spec(block, H),
        out_shape=jax.ShapeDtypeStruct((E2, H), jnp.float32),
    )(g1, g2, w2t, b2, w3t, b3)


def _update_body(agg_ref, st_ref, ff_ref,
                 wih_ref, bih_ref, whh_ref, bhh_ref, wsi_ref, wso_ref,
                 st_out_ref, uv_ref=None):
    state = st_ref[...]
    agg = agg_ref[...]
    gi = _bdot(agg, wih_ref[...]) + bih_ref[...]
    gh = _bdot(state, whh_ref[...]) + bhh_ref[...]
    r = jax.nn.sigmoid(gi[:, 0:H] + gh[:, 0:H])
    z = jax.nn.sigmoid(gi[:, H:2 * H] + gh[:, H:2 * H])
    n = jnp.tanh(gi[:, 2 * H:3 * H] + r * gh[:, 2 * H:3 * H])
    new_state = (1.0 - z) * n + z * state
    st_out_ref[...] = new_state
    if uv_ref is not None:
        sb = new_state.astype(jnp.bfloat16)
        u = jnp.dot(sb, wsi_ref[...], preferred_element_type=jnp.float32)
        v = jnp.dot(sb, wso_ref[...], preferred_element_type=jnp.float32)
        uv_ref[...] = jnp.concatenate([u, v], axis=1) + ff_ref[...]


def _update_call(agg2, state, ff, wiht, bih, whht, bhh,
                 wsit, wsot, with_uv, block=1600):
    E = agg2.shape[0]
    grid = (E // block,)
    out_specs = [_row_spec(block, H)]
    out_shape = [jax.ShapeDtypeStruct((E, H), jnp.float32)]
    if with_uv:
        out_specs += [_row_spec(block, 2 * H)]
        out_shape += [jax.ShapeDtypeStruct((E, 2 * H), jnp.float32)]
    return pl.pallas_call(
        _update_body,
        grid=grid,
        in_specs=[
            _row_spec(block, H),
            _row_spec(block, H),
            _row_spec(block, 2 * H),
            _full_spec((H, 3 * H)),
            _full_spec((1, 3 * H)),
            _full_spec((H, 3 * H)),
            _full_spec((1, 3 * H)),
            _full_spec((H, H)),
            _full_spec((H, H)),
        ],
        out_specs=out_specs,
        out_shape=out_shape,
    )(agg2, state, ff, wiht, bih, whht, bhh, wsit, wsot)


def _out_body(sagg_ref, b_ref, wh_ref, wb_ref, b1_ref, w2_ref, b2_ref,
              w3_ref, b3_ref, o_ref):
    bcol = b_ref[...].astype(jnp.bfloat16).astype(jnp.float32)
    sagg = sagg_ref[...]
    x1 = _bdot(sagg, wh_ref[...]) + bcol * wb_ref[...] + b1_ref[...]
    x1 = jnp.maximum(x1, 0.0)
    x2 = jnp.maximum(_bdot(x1, w2_ref[...]) + b2_ref[...], 0.0)
    y = _bdot(x2, w3_ref[...]) + b3_ref[...]
    m = jnp.max(y, axis=1, keepdims=True)
    lse = jnp.log(jnp.sum(jnp.exp(y - m), axis=1, keepdims=True)) + m
    o_ref[...] = y - lse


def _out_call(sagg, b, wh, wb, b1, w2t, b2, w3t, b3, block=2000):
    N = sagg.shape[0]
    grid = (N // block,)
    return pl.pallas_call(
        _out_body,
        grid=grid,
        in_specs=[
            _row_spec(block, H),
            _row_spec(block, 1),
            _full_spec((H, H)),
            _full_spec((1, H)),
            _full_spec((1, H)),
            _full_spec((H, H)),
            _full_spec((1, H)),
            _full_spec((H, 2)),
            _full_spec((1, 2)),
        ],
        out_specs=_row_spec(block, 2),
        out_shape=jax.ShapeDtypeStruct((N, 2), jnp.float32),
    )(sagg, b, wh, wb, b1, w2t, b2, w3t, b3)


def _sc_gather_call(uv, ein, eout, chunk=400):
    E2 = ein.shape[0]
    per_w = E2 // _NW
    mesh = plsc.VectorSubcoreMesh(**_SC_MESH)

    @functools.partial(
        pl.kernel,
        out_type=[jax.ShapeDtypeStruct((E2, 2 * H), jnp.float32),
                  jax.ShapeDtypeStruct((E2, 2 * H), jnp.float32)],
        mesh=mesh,
        scratch_types=[
            pltpu.VMEM((chunk,), jnp.int32),
            pltpu.VMEM((chunk,), jnp.int32),
            pltpu.VMEM((chunk, 2 * H), jnp.float32),
            pltpu.VMEM((chunk, 2 * H), jnp.float32),
            pltpu.SemaphoreType.DMA,
            pltpu.SemaphoreType.DMA,
        ],
    )
    def gk(uv_hbm, ein_hbm, eout_hbm, g1_hbm, g2_hbm,
           idx1, idx2, r1, r2, s1, s2):
        base0 = _wid() * per_w

        @pl.loop(0, per_w, step=chunk)
        def _(off):
            base = base0 + off
            pltpu.sync_copy(ein_hbm.at[pl.ds(base, chunk)], idx1)
            pltpu.sync_copy(eout_hbm.at[pl.ds(base, chunk)], idx2)
            cp1 = pltpu.async_copy(uv_hbm.at[idx1], r1, s1)
            cp2 = pltpu.async_copy(uv_hbm.at[idx2], r2, s2)
            cp1.wait()
            cp2.wait()
            pltpu.sync_copy(r1, g1_hbm.at[pl.ds(base, chunk)])
            pltpu.sync_copy(r2, g2_hbm.at[pl.ds(base, chunk)])

    return gk(uv, ein, eout)


def _sc_scatter_call(ms, eout, zeros8, E, chunk=200):
    E2 = eout.shape[0]
    per_sc = E2 // _NC
    per_tile = per_sc // _NS
    rows_t = E // _NS
    zrows = zeros8.shape[0]
    mesh = plsc.VectorSubcoreMesh(**_SC_MESH)

    @functools.partial(
        pl.kernel,
        out_type=jax.ShapeDtypeStruct((8, _NC, E, 8), jnp.float32),
        mesh=mesh,
        scratch_types=[
            pltpu.VMEM_SHARED((E, 8), jnp.float32),
            pltpu.VMEM((chunk,), jnp.int32),
            pltpu.VMEM((chunk, 8), jnp.float32),
            pltpu.SemaphoreType.DMA,
        ],
    )
    def sk(m0, m1, m2, m3, m4, m5, m6, m7, eout_hbm, z_hbm, part_hbm,
           acc, idx, mbuf, sem):
        c = lax.axis_index("c")
        s = lax.axis_index("s")
        rbase = s * rows_t
        ebase = c * per_sc + s * per_tile

        for g, mg in enumerate((m0, m1, m2, m3, m4, m5, m6, m7)):
            @pl.loop(0, rows_t, step=zrows)
            def _(z):
                pltpu.sync_copy(z_hbm, acc.at[pl.ds(rbase + z, zrows)])

            plsc.subcore_barrier()

            @pl.loop(0, per_tile, step=chunk)
            def _(off):
                base = ebase + off
                pltpu.sync_copy(eout_hbm.at[pl.ds(base, chunk)], idx)
                pltpu.async_copy(mg.at[pl.ds(base, chunk)], mbuf, sem).wait()
                pltpu.sync_copy(mbuf, acc.at[idx], add=True)

            plsc.subcore_barrier()
            pltpu.sync_copy(
                acc.at[pl.ds(rbase, rows_t)],
                part_hbm.at[g, c, pl.ds(rbase, rows_t)],
            )
            plsc.subcore_barrier()

    return sk(*ms, eout, zeros8)


def _sc_bgather_call(bflat, mn0, mn1, chunk=1000):
    E = mn0.shape[0]
    per_w = E // _NW
    mesh = plsc.VectorSubcoreMesh(**_SC_MESH)

    @functools.partial(
        pl.kernel,
        out_type=[jax.ShapeDtypeStruct((E,), jnp.float32),
                  jax.ShapeDtypeStruct((E,), jnp.float32)],
        mesh=mesh,
        scratch_types=[
            pltpu.VMEM((chunk,), jnp.int32),
            pltpu.VMEM((chunk,), jnp.float32),
            pltpu.SemaphoreType.DMA,
        ],
    )
    def bk(b_hbm, mn0_hbm, mn1_hbm, bi_hbm, bo_hbm, idx, vals, sem):
        base0 = _wid() * per_w

        @pl.loop(0, per_w, step=chunk)
        def _(off):
            base = base0 + off
            pltpu.sync_copy(mn0_hbm.at[pl.ds(base, chunk)], idx)
            pltpu.async_copy(b_hbm.at[idx], vals, sem).wait()
            pltpu.sync_copy(vals, bi_hbm.at[pl.ds(base, chunk)])
            pltpu.sync_copy(mn1_hbm.at[pl.ds(base, chunk)], idx)
            pltpu.async_copy(b_hbm.at[idx], vals, sem).wait()
            pltpu.sync_copy(vals, bo_hbm.at[pl.ds(base, chunk)])

    return bk(bflat, mn0, mn1)


def _sc_nscatter_call(state, mn1, zeros64, N, chunk=200):
    E = mn1.shape[0]
    per_sc = E // _NC
    per_tile = per_sc // _NS
    stride, span = 624, 640
    zrows = zeros64.shape[0]
    mesh = plsc.VectorSubcoreMesh(**_SC_MESH)

    @functools.partial(
        pl.kernel,
        out_type=jax.ShapeDtypeStruct((_NC, N, H), jnp.float32),
        mesh=mesh,
        scratch_types=[
            pltpu.VMEM_SHARED((N, H), jnp.float32),
            pltpu.VMEM((chunk,), jnp.int32),
            pltpu.VMEM((chunk, H), jnp.float32),
            pltpu.SemaphoreType.DMA,
        ],
    )
    def nk(st_hbm, mn1_hbm, z_hbm, part_hbm, acc, idx, sbuf, sem):
        c = lax.axis_index("c")
        s = lax.axis_index("s")
        rbase = s * stride
        ebase = c * per_sc + s * per_tile

        @pl.loop(0, span, step=zrows)
        def _(z):
            pltpu.sync_copy(z_hbm, acc.at[pl.ds(rbase + z, zrows)])

        plsc.subcore_barrier()

        @pl.loop(0, per_tile, step=chunk)
        def _(off):
            base = ebase + off
            pltpu.sync_copy(mn1_hbm.at[pl.ds(base, chunk)], idx)
            pltpu.async_copy(st_hbm.at[pl.ds(base, chunk)], sbuf, sem).wait()
            pltpu.sync_copy(sbuf, acc.at[idx], add=True)

        plsc.subcore_barrier()
        pltpu.sync_copy(acc.at[pl.ds(rbase, span)],
                        part_hbm.at[c, pl.ds(rbase, span)])

    return nk(state, mn1, zeros64)


def _b16(x):
    return x.astype(jnp.bfloat16)


def kernel(J_msg, b, msg_node, idx_msg_edge, params):
    N = b.shape[0]
    E = msg_node.shape[0]
    p = params

    W1b = _b16(p['msg_W1'])
    W1f = W1b.astype(jnp.float32)
    wsit = W1b[:, 0:H].T
    wsot = W1b[:, H + 8:2 * H + 8].T
    Wfi = W1f[:, H:H + 8]
    Wfo = W1f[:, 2 * H + 8:2 * H + 16]
    av = jnp.stack([Wfi[:, 0] - Wfi[:, 1],
                    Wfi[:, 2] - Wfi[:, 3],
                    Wfi[:, 4] - Wfi[:, 5] - Wfi[:, 6] + Wfi[:, 7]])
    dv = jnp.stack([Wfo[:, 0] - Wfo[:, 1],
                    Wfo[:, 2] - Wfo[:, 3],
                    Wfo[:, 4] - Wfo[:, 5] - Wfo[:, 6] + Wfo[:, 7],
                    p['msg_b1']])
    w2t = _b16(p['msg_W2']).T
    b2 = p['msg_b2'].reshape(1, H)
    w3t = _b16(p['msg_W3']).T
    b3 = p['msg_b3'].reshape(1, H)
    wiht = _b16(p['gru_Wih']).T
    whht = _b16(p['gru_Whh']).T
    bih = p['gru_bih'].reshape(1, 3 * H)
    bhh = p['gru_bhh'].reshape(1, 3 * H)
    OW1 = _b16(p['out_W1'])
    wh = OW1[:, 0:H].T
    wb = (OW1[:, H].astype(jnp.float32)
          - OW1[:, H + 1].astype(jnp.float32)).reshape(1, H)
    ob1 = p['out_b1'].reshape(1, H)
    ow2t = _b16(p['out_W2']).T
    ob2 = p['out_b2'].reshape(1, H)
    ow3t = _b16(p['out_W3']).T
    ob3 = p['out_b3'].reshape(1, 2)

    mn0 = jnp.asarray(msg_node[:, 0]).copy()
    mn1 = jnp.asarray(msg_node[:, 1]).copy()
    ein = jnp.asarray(idx_msg_edge[:, 0]).copy()
    eout = jnp.asarray(idx_msg_edge[:, 1]).copy()
    bflat = jnp.asarray(b[:, 0]).copy()
    zeros8 = jnp.zeros((2000, 8), jnp.float32)
    zeros64 = jnp.zeros((128, H), jnp.float32)

    bi, bo = bflat[mn0], bflat[mn1]

    ff = _prep_call(bi.reshape(E, 1), bo.reshape(E, 1), J_msg, av, dv)

    state = jnp.zeros((E, H), jnp.float32)
    uv = ff
    for step in range(NUM_PROP):
        g1, g2 = uv[ein], uv[eout]
        m = _mid_call(g1, g2, w2t, b2, w3t, b3)
        agg = jnp.zeros((E, H), jnp.float32).at[eout].add(m)
        outs = _update_call(agg, state, ff, wiht, bih,
                            whht, bhh, wsit, wsot,
                            with_uv=(step < NUM_PROP - 1))
        if step < NUM_PROP - 1:
            state, uv = outs
        else:
            state = outs[0]

    sagg = jnp.zeros((N, H), jnp.float32).at[mn1].add(state)
    return _out_call(sagg, b, wh, wb, ob1, ow2t, ob2, ow3t, ob3)


# speedup vs baseline: 11.4727x; 1.0004x over previous
import functools

import jax
import jax.numpy as jnp
from jax import lax
from jax.experimental import pallas as pl
from jax.experimental.pallas import tpu as pltpu
from jax.experimental.pallas import tpu_sc as plsc

H = 64
NUM_PROP = 3

_NC, _NS = 2, 16
_NW = _NC * _NS

_SC_MESH = dict(core_axis_name="c", subcore_axis_name="s")


def _wid():
    return lax.axis_index("s") * _NC + lax.axis_index("c")


def _row_spec(rows, cols):
    return pl.BlockSpec((rows, cols), lambda i: (i, 0))


def _full_spec(shape):
    return pl.BlockSpec(shape, lambda i: (0,) * len(shape))


def _bdot(a, b):
    return jnp.dot(a.astype(jnp.bfloat16), b,
                   preferred_element_type=jnp.float32)


def _prep_body(bi_ref, bo_ref, jm_ref, av_ref, dv_ref, fa_ref):
    bi = bi_ref[...].astype(jnp.bfloat16).astype(jnp.float32)
    bo = bo_ref[...].astype(jnp.bfloat16).astype(jnp.float32)
    jm = jm_ref[...].astype(jnp.bfloat16).astype(jnp.float32)
    av = av_ref[...]
    dv = dv_ref[...]
    ffa = bi * av[0:1, :] + bo * av[1:2, :] + jm * av[2:3, :]
    ffb = bi * dv[0:1, :] + bo * dv[1:2, :] + jm * dv[2:3, :] + dv[3:4, :]
    fa_ref[...] = jnp.concatenate([ffa, ffb], axis=1)


def _prep_call(bi, bo, jm, av, dv, block=8000):
    E = bi.shape[0]
    block = min(block, E)
    grid = (E // block,)
    return pl.pallas_call(
        _prep_body,
        grid=grid,
        in_specs=[
            _row_spec(block, 1),
            _row_spec(block, 1),
            _row_spec(block, 1),
            _full_spec((3, H)),
            _full_spec((4, H)),
        ],
        out_specs=_row_spec(block, 2 * H),
        out_shape=jax.ShapeDtypeStruct((E, 2 * H), jnp.float32),
    )(bi, bo, jm, av, dv)


def _mid_body(g1_ref, g2_ref, w2_ref, b2_ref, w3_ref, b3_ref, m_ref):
    h1 = jnp.maximum(g1_ref[:, 0:H] + g2_ref[:, H:2 * H], 0.0)
    h2 = jnp.maximum(_bdot(h1, w2_ref[...]) + b2_ref[...], 0.0)
    m_ref[...] = _bdot(h2, w3_ref[...]) + b3_ref[...]


def _mid_body_g(g1_ref, g2_ref, w2_ref, b2_ref, w3_ref, b3_ref, *m_refs):
    h1 = jnp.maximum(g1_ref[:, 0:H] + g2_ref[:, H:2 * H], 0.0)
    h2 = jnp.maximum(_bdot(h1, w2_ref[...]) + b2_ref[...], 0.0)
    m = _bdot(h2, w3_ref[...]) + b3_ref[...]
    for g in range(8):
        m_refs[g][...] = m[:, 8 * g:8 * (g + 1)]


def _mid_call_g(g1, g2, w2t, b2, w3t, b3, block=3200):
    E2 = g1.shape[0]
    grid = (E2 // block,)
    return pl.pallas_call(
        _mid_body_g,
        grid=grid,
        in_specs=[
            _row_spec(block, 2 * H),
            _row_spec(block, 2 * H),
            _full_spec((H, H)),
            _full_spec((1, H)),
            _full_spec((H, H)),
            _full_spec((1, H)),
        ],
        out_specs=[_row_spec(block, 8)] * 8,
        out_shape=[jax.ShapeDtypeStruct((E2, 8), jnp.float32)] * 8,
    )(g1, g2, w2t, b2, w3t, b3)


def _mid_call(g1, g2, w2t, b2, w3t, b3, block=3200):
    E2 = g1.shape[0]
    grid = (E2 // block,)
    return pl.pallas_call(
        _mid_body,
        grid=grid,
        in_specs=[
            _row_spec(block, 2 * H),
            _row_spec(block, 2 * H),
            _full_spec((H, H)),
            _full_spec((1, H)),
            _full_spec((H, H)),
            _full_spec((1, H)),
        ],
        out_specs=_row_spec(block, H),
        out_shape=jax.ShapeDtypeStruct((E2, H), jnp.float32),
    )(g1, g2, w2t, b2, w3t, b3)


def _update_body_g(agg_ref, st_ref, ff_ref,
                   wih_ref, bih_ref, whh_ref, bhh_ref, wsi_ref, wso_ref,
                   st_out_ref, uv_ref=None):
    state = st_ref[...]
    agg = jnp.concatenate(
        [agg_ref[g, 0] + agg_ref[g, 1] for g in range(8)], axis=1)
    gi = _bdot(agg, wih_ref[...]) + bih_ref[...]
    gh = _bdot(state, whh_ref[...]) + bhh_ref[...]
    r = jax.nn.sigmoid(gi[:, 0:H] + gh[:, 0:H])
    z = jax.nn.sigmoid(gi[:, H:2 * H] + gh[:, H:2 * H])
    n = jnp.tanh(gi[:, 2 * H:3 * H] + r * gh[:, 2 * H:3 * H])
    new_state = (1.0 - z) * n + z * state
    st_out_ref[...] = new_state
    if uv_ref is not None:
        sb = new_state.astype(jnp.bfloat16)
        u = jnp.dot(sb, wsi_ref[...], preferred_element_type=jnp.float32)
        v = jnp.dot(sb, wso_ref[...], preferred_element_type=jnp.float32)
        uv_ref[...] = jnp.concatenate([u, v], axis=1) + ff_ref[...]


def _update_call_g(agg2, state, ff, wiht, bih, whht, bhh,
                   wsit, wsot, with_uv, block=1600):
    E = agg2.shape[2]
    grid = (E // block,)
    out_specs = [_row_spec(block, H)]
    out_shape = [jax.ShapeDtypeStruct((E, H), jnp.float32)]
    if with_uv:
        out_specs += [_row_spec(block, 2 * H)]
        out_shape += [jax.ShapeDtypeStruct((E, 2 * H), jnp.float32)]
    return pl.pallas_call(
        _update_body_g,
        grid=grid,
        in_specs=[
            pl.BlockSpec((8, 2, block, 8), lambda i: (0, 0, i, 0)),
            _row_spec(block, H),
            _row_spec(block, 2 * H),
            _full_spec((H, 3 * H)),
            _full_spec((1, 3 * H)),
            _full_spec((H, 3 * H)),
            _full_spec((1, 3 * H)),
            _full_spec((H, H)),
            _full_spec((H, H)),
        ],
        out_specs=out_specs,
        out_shape=out_shape,
    )(agg2, state, ff, wiht, bih, whht, bhh, wsit, wsot)


def _update_body(agg_ref, st_ref, ff_ref,
                 wih_ref, bih_ref, whh_ref, bhh_ref, wsi_ref, wso_ref,
                 st_out_ref, uv_ref=None):
    state = st_ref[...]
    agg = agg_ref[...]
    gi = _bdot(agg, wih_ref[...]) + bih_ref[...]
    gh = _bdot(state, whh_ref[...]) + bhh_ref[...]
    r = jax.nn.sigmoid(gi[:, 0:H] + gh[:, 0:H])
    z = jax.nn.sigmoid(gi[:, H:2 * H] + gh[:, H:2 * H])
    n = jnp.tanh(gi[:, 2 * H:3 * H] + r * gh[:, 2 * H:3 * H])
    new_state = (1.0 - z) * n + z * state
    st_out_ref[...] = new_state
    if uv_ref is not None:
        sb = new_state.astype(jnp.bfloat16)
        u = jnp.dot(sb, wsi_ref[...], preferred_element_type=jnp.float32)
        v = jnp.dot(sb, wso_ref[...], preferred_element_type=jnp.float32)
        uv_ref[...] = jnp.concatenate([u, v], axis=1) + ff_ref[...]


def _update_call(agg2, state, ff, wiht, bih, whht, bhh,
                 wsit, wsot, with_uv, block=1600):
    E = agg2.shape[0]
    grid = (E // block,)
    out_specs = [_row_spec(block, H)]
    out_shape = [jax.ShapeDtypeStruct((E, H), jnp.float32)]
    if with_uv:
        out_specs += [_row_spec(block, 2 * H)]
        out_shape += [jax.ShapeDtypeStruct((E, 2 * H), jnp.float32)]
    return pl.pallas_call(
        _update_body,
        grid=grid,
        in_specs=[
            _row_spec(block, H),
            _row_spec(block, H),
            _row_spec(block, 2 * H),
            _full_spec((H, 3 * H)),
            _full_spec((1, 3 * H)),
            _full_spec((H, 3 * H)),
            _full_spec((1, 3 * H)),
            _full_spec((H, H)),
            _full_spec((H, H)),
        ],
        out_specs=out_specs,
        out_shape=out_shape,
    )(agg2, state, ff, wiht, bih, whht, bhh, wsit, wsot)


def _out_body(sagg_ref, b_ref, wh_ref, wb_ref, b1_ref, w2_ref, b2_ref,
              w3_ref, b3_ref, o_ref):
    bcol = b_ref[...].astype(jnp.bfloat16).astype(jnp.float32)
    sagg = sagg_ref[...]
    x1 = _bdot(sagg, wh_ref[...]) + bcol * wb_ref[...] + b1_ref[...]
    x1 = jnp.maximum(x1, 0.0)
    x2 = jnp.maximum(_bdot(x1, w2_ref[...]) + b2_ref[...], 0.0)
    y = _bdot(x2, w3_ref[...]) + b3_ref[...]
    m = jnp.max(y, axis=1, keepdims=True)
    lse = jnp.log(jnp.sum(jnp.exp(y - m), axis=1, keepdims=True)) + m
    o_ref[...] = y - lse


def _out_call(sagg, b, wh, wb, b1, w2t, b2, w3t, b3, block=2000):
    N = sagg.shape[0]
    grid = (N // block,)
    return pl.pallas_call(
        _out_body,
        grid=grid,
        in_specs=[
            _row_spec(block, H),
            _row_spec(block, 1),
            _full_spec((H, H)),
            _full_spec((1, H)),
            _full_spec((1, H)),
            _full_spec((H, H)),
            _full_spec((1, H)),
            _full_spec((H, 2)),
            _full_spec((1, 2)),
        ],
        out_specs=_row_spec(block, 2),
        out_shape=jax.ShapeDtypeStruct((N, 2), jnp.float32),
    )(sagg, b, wh, wb, b1, w2t, b2, w3t, b3)


def _sc_gather_call(uv, ein, eout, chunk=128):
    E2 = ein.shape[0]
    per_w = E2 // _NW
    main = per_w // chunk * chunk
    tail = per_w - main
    mesh = plsc.VectorSubcoreMesh(**_SC_MESH)

    @functools.partial(
        pl.kernel,
        out_type=[jax.ShapeDtypeStruct((E2, 2 * H), jnp.float32),
                  jax.ShapeDtypeStruct((E2, 2 * H), jnp.float32)],
        mesh=mesh,
        scratch_types=[
            pltpu.VMEM((chunk,), jnp.int32),
            pltpu.VMEM((chunk,), jnp.int32),
            pltpu.VMEM((chunk, 2 * H), jnp.float32),
            pltpu.VMEM((chunk, 2 * H), jnp.float32),
            pltpu.VMEM((tail,), jnp.int32),
            pltpu.VMEM((tail,), jnp.int32),
            pltpu.VMEM((tail, 2 * H), jnp.float32),
            pltpu.VMEM((tail, 2 * H), jnp.float32),
            pltpu.SemaphoreType.DMA,
            pltpu.SemaphoreType.DMA,
        ],
    )
    def gk(uv_hbm, ein_hbm, eout_hbm, g1_hbm, g2_hbm,
           idx1, idx2, r1, r2, ti1, ti2, tr1, tr2, s1, s2):
        base0 = _wid() * per_w

        def step(base, i1, i2, b1, b2, n):
            pltpu.sync_copy(ein_hbm.at[pl.ds(base, n)], i1)
            pltpu.sync_copy(eout_hbm.at[pl.ds(base, n)], i2)
            cp1 = pltpu.async_copy(uv_hbm.at[i1], b1, s1)
            cp2 = pltpu.async_copy(uv_hbm.at[i2], b2, s2)
            cp1.wait()
            cp2.wait()
            pltpu.sync_copy(b1, g1_hbm.at[pl.ds(base, n)])
            pltpu.sync_copy(b2, g2_hbm.at[pl.ds(base, n)])

        @pl.loop(0, main, step=chunk)
        def _(off):
            step(base0 + off, idx1, idx2, r1, r2, chunk)

        if tail:
            step(base0 + main, ti1, ti2, tr1, tr2, tail)

    return gk(uv, ein, eout)


def _sc_scatter_call(ms, eout, zeros8, E, chunk=128):
    E2 = eout.shape[0]
    per_sc = E2 // _NC
    per_tile = per_sc // _NS
    main = per_tile // chunk * chunk
    tail = per_tile - main
    rows_t = E // _NS
    zrows = zeros8.shape[0]
    mesh = plsc.VectorSubcoreMesh(**_SC_MESH)

    @functools.partial(
        pl.kernel,
        out_type=jax.ShapeDtypeStruct((8, _NC, E, 8), jnp.float32),
        mesh=mesh,
        scratch_types=[
            pltpu.VMEM_SHARED((E, 8), jnp.float32),
            pltpu.VMEM((chunk,), jnp.int32),
            pltpu.VMEM((chunk, 8), jnp.float32),
            pltpu.VMEM((tail,), jnp.int32),
            pltpu.VMEM((tail, 8), jnp.float32),
            pltpu.SemaphoreType.DMA,
        ],
    )
    def sk(m0, m1, m2, m3, m4, m5, m6, m7, eout_hbm, z_hbm, part_hbm,
           acc, idx, mbuf, tidx, tmbuf, sem):
        c = lax.axis_index("c")
        s = lax.axis_index("s")
        rbase = s * rows_t
        ebase = c * per_sc + s * per_tile

        for g, mg in enumerate((m0, m1, m2, m3, m4, m5, m6, m7)):
            @pl.loop(0, rows_t, step=zrows)
            def _(z):
                pltpu.sync_copy(z_hbm, acc.at[pl.ds(rbase + z, zrows)])

            plsc.subcore_barrier()

            def step(base, ib, mb, n):
                pltpu.sync_copy(eout_hbm.at[pl.ds(base, n)], ib)
                pltpu.async_copy(mg.at[pl.ds(base, n)], mb, sem).wait()
                pltpu.sync_copy(mb, acc.at[ib], add=True)

            @pl.loop(0, main, step=chunk)
            def _(off):
                step(ebase + off, idx, mbuf, chunk)

            if tail:
                step(ebase + main, tidx, tmbuf, tail)

            plsc.subcore_barrier()
            pltpu.sync_copy(
                acc.at[pl.ds(rbase, rows_t)],
                part_hbm.at[g, c, pl.ds(rbase, rows_t)],
            )
            plsc.subcore_barrier()

    return sk(*ms, eout, zeros8)


def _sc_bgather_call(bflat, mn0, mn1, chunk=128):
    E = mn0.shape[0]
    per_w = E // _NW
    main = per_w // chunk * chunk
    tail = per_w - main
    mesh = plsc.VectorSubcoreMesh(**_SC_MESH)

    @functools.partial(
        pl.kernel,
        out_type=[jax.ShapeDtypeStruct((E,), jnp.float32),
                  jax.ShapeDtypeStruct((E,), jnp.float32)],
        mesh=mesh,
        scratch_types=[
            pltpu.VMEM((chunk,), jnp.int32),
            pltpu.VMEM((chunk,), jnp.float32),
            pltpu.VMEM((tail,), jnp.int32),
            pltpu.VMEM((tail,), jnp.float32),
            pltpu.SemaphoreType.DMA,
        ],
    )
    def bk(b_hbm, mn0_hbm, mn1_hbm, bi_hbm, bo_hbm,
           idx, vals, tidx, tvals, sem):
        base0 = _wid() * per_w

        def step(base, ib, vb, n):
            pltpu.sync_copy(mn0_hbm.at[pl.ds(base, n)], ib)
            pltpu.async_copy(b_hbm.at[ib], vb, sem).wait()
            pltpu.sync_copy(vb, bi_hbm.at[pl.ds(base, n)])
            pltpu.sync_copy(mn1_hbm.at[pl.ds(base, n)], ib)
            pltpu.async_copy(b_hbm.at[ib], vb, sem).wait()
            pltpu.sync_copy(vb, bo_hbm.at[pl.ds(base, n)])

        @pl.loop(0, main, step=chunk)
        def _(off):
            step(base0 + off, idx, vals, chunk)

        if tail:
            step(base0 + main, tidx, tvals, tail)

    return bk(bflat, mn0, mn1)


def _sc_nscatter_call(state, mn1, zeros64, N, chunk=128):
    E = mn1.shape[0]
    per_sc = E // _NC
    per_tile = per_sc // _NS
    main = per_tile // chunk * chunk
    tail = per_tile - main
    stride, span = 624, 640
    zrows = zeros64.shape[0]
    mesh = plsc.VectorSubcoreMesh(**_SC_MESH)

    @functools.partial(
        pl.kernel,
        out_type=jax.ShapeDtypeStruct((_NC, N, H), jnp.float32),
        mesh=mesh,
        scratch_types=[
            pltpu.VMEM_SHARED((N, H), jnp.float32),
            pltpu.VMEM((chunk,), jnp.int32),
            pltpu.VMEM((chunk, H), jnp.float32),
            pltpu.VMEM((tail,), jnp.int32),
            pltpu.VMEM((tail, H), jnp.float32),
            pltpu.SemaphoreType.DMA,
        ],
    )
    def nk(st_hbm, mn1_hbm, z_hbm, part_hbm, acc, idx, sbuf,
           tidx, tsbuf, sem):
        c = lax.axis_index("c")
        s = lax.axis_index("s")
        rbase = s * stride
        ebase = c * per_sc + s * per_tile

        @pl.loop(0, span, step=zrows)
        def _(z):
            pltpu.sync_copy(z_hbm, acc.at[pl.ds(rbase + z, zrows)])

        plsc.subcore_barrier()

        def step(base, ib, sb, n):
            pltpu.sync_copy(mn1_hbm.at[pl.ds(base, n)], ib)
            pltpu.async_copy(st_hbm.at[pl.ds(base, n)], sb, sem).wait()
            pltpu.sync_copy(sb, acc.at[ib], add=True)

        @pl.loop(0, main, step=chunk)
        def _(off):
            step(ebase + off, idx, sbuf, chunk)

        if tail:
            step(ebase + main, tidx, tsbuf, tail)

        plsc.subcore_barrier()
        pltpu.sync_copy(acc.at[pl.ds(rbase, span)],
                        part_hbm.at[c, pl.ds(rbase, span)])

    return nk(state, mn1, zeros64)


def _b16(x):
    return x.astype(jnp.bfloat16)


def kernel(J_msg, b, msg_node, idx_msg_edge, params):
    N = b.shape[0]
    E = msg_node.shape[0]
    p = params

    W1b = _b16(p['msg_W1'])
    W1f = W1b.astype(jnp.float32)
    wsit = W1b[:, 0:H].T
    wsot = W1b[:, H + 8:2 * H + 8].T
    Wfi = W1f[:, H:H + 8]
    Wfo = W1f[:, 2 * H + 8:2 * H + 16]
    av = jnp.stack([Wfi[:, 0] - Wfi[:, 1],
                    Wfi[:, 2] - Wfi[:, 3],
                    Wfi[:, 4] - Wfi[:, 5] - Wfi[:, 6] + Wfi[:, 7]])
    dv = jnp.stack([Wfo[:, 0] - Wfo[:, 1],
                    Wfo[:, 2] - Wfo[:, 3],
                    Wfo[:, 4] - Wfo[:, 5] - Wfo[:, 6] + Wfo[:, 7],
                    p['msg_b1']])
    w2t = _b16(p['msg_W2']).T
    b2 = p['msg_b2'].reshape(1, H)
    w3t = _b16(p['msg_W3']).T
    b3 = p['msg_b3'].reshape(1, H)
    wiht = _b16(p['gru_Wih']).T
    whht = _b16(p['gru_Whh']).T
    bih = p['gru_bih'].reshape(1, 3 * H)
    bhh = p['gru_bhh'].reshape(1, 3 * H)
    OW1 = _b16(p['out_W1'])
    wh = OW1[:, 0:H].T
    wb = (OW1[:, H].astype(jnp.float32)
          - OW1[:, H + 1].astype(jnp.float32)).reshape(1, H)
    ob1 = p['out_b1'].reshape(1, H)
    ow2t = _b16(p['out_W2']).T
    ob2 = p['out_b2'].reshape(1, H)
    ow3t = _b16(p['out_W3']).T
    ob3 = p['out_b3'].reshape(1, 2)

    mn0 = jnp.asarray(msg_node[:, 0]).copy()
    mn1 = jnp.asarray(msg_node[:, 1]).copy()
    ein = jnp.asarray(idx_msg_edge[:, 0]).copy()
    eout = jnp.asarray(idx_msg_edge[:, 1]).copy()
    bflat = jnp.asarray(b[:, 0]).copy()
    zeros8 = jnp.zeros((2000, 8), jnp.float32)
    zeros64 = jnp.zeros((128, H), jnp.float32)

    bi, bo = bflat[mn0], bflat[mn1]

    ff = _prep_call(bi.reshape(E, 1), bo.reshape(E, 1), J_msg, av, dv)

    state = jnp.zeros((E, H), jnp.float32)
    uv = ff
    for step in range(NUM_PROP):
        g1, g2 = uv[ein], uv[eout]
        ms = _mid_call(g1, g2, w2t, b2, w3t, b3)
        agg = jnp.zeros((E, H), jnp.float32).at[eout].add(ms)
        outs = _update_call(agg, state, ff, wiht, bih,
                            whht, bhh, wsit, wsot,
                            with_uv=(step < NUM_PROP - 1))
        if step < NUM_PROP - 1:
            state, uv = outs
        else:
            state = outs[0]

    sagg = jnp.zeros((N, H), jnp.float32).at[mn1].add(state)
    return _out_call(sagg, b, wh, wb, ob1, ow2t, ob2, ow3t, ob3)
